# baseline (speedup 1.0000x reference)
import jax
import jax.numpy as jnp
from jax import lax
from jax.experimental import pallas as pl
from jax.experimental.pallas import tpu as pltpu
from jax.experimental.pallas import tpu_sc as plsc

_BATCH = 4
_SEQ = 2048
_DIM = 1024
_NC = 2
_NS = 16
_NW = _NC * _NS
_ROWS_PER_W = _SEQ // _NW
_CH = 16
_NCHUNK = _ROWS_PER_W // _CH
_CHW = _CH * _DIM
_VEC = 16
_UNROLL = 8


def _sc_body(in_hbm, tab_hbm, out_hbm,
             tbuf0, tbuf1, ibuf0, ibuf1, obuf0, obuf1,
             tsem0, tsem1, lsem0, lsem1, ssem0, ssem1):
    wid = lax.axis_index("s") * _NC + lax.axis_index("c")
    row0 = wid * _ROWS_PER_W

    tbufs = (tbuf0, tbuf1)
    ibufs = (ibuf0, ibuf1)
    obufs = (obuf0, obuf1)
    tsems = (tsem0, tsem1)
    lsems = (lsem0, lsem1)
    ssems = (ssem0, ssem1)

    def tab_off(c):
        return (row0 + c * _CH) * _DIM

    def io_off(c, b):
        return (b * _SEQ + row0 + c * _CH) * _DIM

    tcopies = [None] * _NCHUNK
    tcopies[0] = pltpu.async_copy(
        tab_hbm.at[pl.ds(tab_off(0), _CHW)], tbufs[0], tsems[0])

    items = [(c, b) for c in range(_NCHUNK) for b in range(_BATCH)]
    n_items = len(items)
    lcopies = [None] * n_items
    scopies = [None] * n_items
    lcopies[0] = pltpu.async_copy(
        in_hbm.at[pl.ds(io_off(0, 0), _CHW)], ibufs[0], lsems[0])
    lcopies[1] = pltpu.async_copy(
        in_hbm.at[pl.ds(io_off(0, 1), _CHW)], ibufs[1], lsems[1])

    for k, (c, b) in enumerate(items):
        p = k % 2
        ibuf = ibufs[p]
        obuf = obufs[p]
        tbuf = tbufs[c % 2]

        if k % _BATCH == 0:
            tcopies[c].wait()
            if c + 1 < _NCHUNK:
                tcopies[c + 1] = pltpu.async_copy(
                    tab_hbm.at[pl.ds(tab_off(c + 1), _CHW)],
                    tbufs[(c + 1) % 2], tsems[(c + 1) % 2])

        if k >= 2:
            scopies[k - 2].wait()
        lcopies[k].wait()

        def add_body(i, _, ibuf=ibuf, tbuf=tbuf, obuf=obuf):
            for u in range(_UNROLL):
                sl = pl.ds((i * _UNROLL + u) * _VEC, _VEC)
                obuf[sl] = ibuf[sl] + tbuf[sl]
            return 0

        lax.fori_loop(0, _CHW // (_VEC * _UNROLL), add_body, 0,
                      unroll=False)

        nk = k + 2
        if nk < n_items:
            ncb = items[nk]
            lcopies[nk] = pltpu.async_copy(
                in_hbm.at[pl.ds(io_off(ncb[0], ncb[1]), _CHW)],
                ibufs[p], lsems[p])

        scopies[k] = pltpu.async_copy(
            obuf, out_hbm.at[pl.ds(io_off(c, b), _CHW)], ssems[p])

    scopies[n_items - 2].wait()
    scopies[n_items - 1].wait()


def kernel(inputs, pos_table):
    batch, seq_len, out_dim = inputs.shape
    in_flat = inputs.reshape(batch * seq_len * out_dim)
    tab_flat = pos_table.reshape(seq_len * out_dim)

    mesh = plsc.VectorSubcoreMesh(core_axis_name="c", subcore_axis_name="s")
    sc = pl.kernel(
        _sc_body,
        mesh=mesh,
        out_type=jax.ShapeDtypeStruct((batch * seq_len * out_dim,),
                                      jnp.float32),
        scratch_types=[
            pltpu.VMEM((_CHW,), jnp.float32),
            pltpu.VMEM((_CHW,), jnp.float32),
            pltpu.VMEM((_CHW,), jnp.float32),
            pltpu.VMEM((_CHW,), jnp.float32),
            pltpu.VMEM((_CHW,), jnp.float32),
            pltpu.VMEM((_CHW,), jnp.float32),
            pltpu.SemaphoreType.DMA,
            pltpu.SemaphoreType.DMA,
            pltpu.SemaphoreType.DMA,
            pltpu.SemaphoreType.DMA,
            pltpu.SemaphoreType.DMA,
            pltpu.SemaphoreType.DMA,
        ],
    )
    out_flat = sc(in_flat, tab_flat)
    return out_flat.reshape(batch, seq_len, out_dim)


# speedup vs baseline: 1.0252x; 1.0252x over previous
import jax
import jax.numpy as jnp
from jax import lax
from jax.experimental import pallas as pl
from jax.experimental.pallas import tpu as pltpu
from jax.experimental.pallas import tpu_sc as plsc

_BATCH = 4
_SEQ = 2048
_DIM = 1024
_NC = 2
_NS = 16
_NW = _NC * _NS
_ROWS_PER_W = _SEQ // _NW
_CH = 16
_NCHUNK = _ROWS_PER_W // _CH
_VEC = 16
_SLICES = _DIM // _VEC
_UNROLL = 8


def _sc_body(in_hbm, tab_hbm, out_hbm,
             tbuf0, tbuf1, ibuf0, ibuf1, obuf0, obuf1,
             tsem0, tsem1, lsem0, lsem1, ssem0, ssem1):
    wid = lax.axis_index("s") * _NC + lax.axis_index("c")
    row0 = wid * _ROWS_PER_W

    tbufs = (tbuf0, tbuf1)
    ibufs = (ibuf0, ibuf1)
    obufs = (obuf0, obuf1)
    tsems = (tsem0, tsem1)
    lsems = (lsem0, lsem1)
    ssems = (ssem0, ssem1)

    tcopies = [None] * _NCHUNK
    tcopies[0] = pltpu.async_copy(
        tab_hbm.at[pl.ds(row0, _CH), :], tbufs[0], tsems[0])

    items = [(c, b) for c in range(_NCHUNK) for b in range(_BATCH)]
    n_items = len(items)
    lcopies = [None] * n_items
    scopies = [None] * n_items
    lcopies[0] = pltpu.async_copy(
        in_hbm.at[0, pl.ds(row0, _CH), :], ibufs[0], lsems[0])
    lcopies[1] = pltpu.async_copy(
        in_hbm.at[1, pl.ds(row0, _CH), :], ibufs[1], lsems[1])

    for k, (c, b) in enumerate(items):
        p = k % 2
        ibuf = ibufs[p]
        obuf = obufs[p]
        tbuf = tbufs[c % 2]

        if k % _BATCH == 0:
            tcopies[c].wait()
            if c + 1 < _NCHUNK:
                tcopies[c + 1] = pltpu.async_copy(
                    tab_hbm.at[pl.ds(row0 + (c + 1) * _CH, _CH), :],
                    tbufs[(c + 1) % 2], tsems[(c + 1) % 2])

        if k >= 2:
            scopies[k - 2].wait()
        lcopies[k].wait()

        def row_body(r, _, ibuf=ibuf, tbuf=tbuf, obuf=obuf):
            def col_body(j, _):
                for u in range(_UNROLL):
                    sl = pl.ds((j * _UNROLL + u) * _VEC, _VEC)
                    obuf[r, sl] = ibuf[r, sl] + tbuf[r, sl]
                return 0
            lax.fori_loop(0, _SLICES // _UNROLL, col_body, 0, unroll=False)
            return 0

        lax.fori_loop(0, _CH, row_body, 0, unroll=False)

        nk = k + 2
        if nk < n_items:
            ncb = items[nk]
            lcopies[nk] = pltpu.async_copy(
                in_hbm.at[ncb[1], pl.ds(row0 + ncb[0] * _CH, _CH), :],
                ibufs[p], lsems[p])

        scopies[k] = pltpu.async_copy(
            obuf, out_hbm.at[b, pl.ds(row0 + c * _CH, _CH), :], ssems[p])

    scopies[n_items - 2].wait()
    scopies[n_items - 1].wait()


def kernel(inputs, pos_table):
    batch, seq_len, out_dim = inputs.shape
    mesh = plsc.VectorSubcoreMesh(core_axis_name="c", subcore_axis_name="s")
    sc = pl.kernel(
        _sc_body,
        mesh=mesh,
        out_type=jax.ShapeDtypeStruct((batch, seq_len, out_dim),
                                      jnp.float32),
        scratch_types=[
            pltpu.VMEM((_CH, _DIM), jnp.float32),
            pltpu.VMEM((_CH, _DIM), jnp.float32),
            pltpu.VMEM((_CH, _DIM), jnp.float32),
            pltpu.VMEM((_CH, _DIM), jnp.float32),
            pltpu.VMEM((_CH, _DIM), jnp.float32),
            pltpu.VMEM((_CH, _DIM), jnp.float32),
            pltpu.SemaphoreType.DMA,
            pltpu.SemaphoreType.DMA,
            pltpu.SemaphoreType.DMA,
            pltpu.SemaphoreType.DMA,
            pltpu.SemaphoreType.DMA,
            pltpu.SemaphoreType.DMA,
        ],
    )
    return sc(inputs, pos_table)


# speedup vs baseline: 4.9464x; 4.8248x over previous
import jax
import jax.numpy as jnp
from jax.experimental import pallas as pl

_SEQ_BLOCK = 256


def _add_kernel(in_ref, table_ref, out_ref):
    out_ref[...] = in_ref[...] + table_ref[...][None, :, :]


def kernel(inputs, pos_table):
    batch, seq_len, out_dim = inputs.shape
    grid = (seq_len // _SEQ_BLOCK,)
    return pl.pallas_call(
        _add_kernel,
        grid=grid,
        in_specs=[
            pl.BlockSpec((batch, _SEQ_BLOCK, out_dim), lambda i: (0, i, 0)),
            pl.BlockSpec((_SEQ_BLOCK, out_dim), lambda i: (i, 0)),
        ],
        out_specs=pl.BlockSpec((batch, _SEQ_BLOCK, out_dim), lambda i: (0, i, 0)),
        out_shape=jax.ShapeDtypeStruct(inputs.shape, inputs.dtype),
    )(inputs, pos_table)
